# baseline (speedup 1.0000x reference)
import functools
import jax
import jax.numpy as jnp
from jax import lax
from jax.experimental import pallas as pl
from jax.experimental.pallas import tpu as pltpu
from jax.experimental.pallas import tpu_sc as plsc

N = 2048
E = 32768
D = 8

_RB = 256
_EB = 512

_NC = 2
_NS = 16
_L = 16
_NW = _NC * _NS
_ROWS_PER_TILE = 32
_PASSES = N // (_NW * _ROWS_PER_TILE)
_ECHUNK = 4096


def _rsqrt16(x):
    i = plsc.bitcast(x, jnp.int32)
    y = plsc.bitcast(jnp.int32(0x5F3759DF) - (i >> 1), jnp.float32)
    for _ in range(3):
        y = y * (1.5 - 0.5 * x * y * y)
    return y


def _unrolled(n_vecs, body16, unroll=4):
    def _body(k, _):
        for u in range(unroll):
            body16(k * unroll + u)
        return 0
    lax.fori_loop(0, n_vecs // unroll, _body, 0)


def _adj_body(row_hbm, col_hbm, ew_hbm, z_hbm, a_hbm,
              deg_l, dinv_l, a_chunk,
              cb0, wb0, rb0, cb1, wb1, rb1,
              sem0, sem1, zsem):
    cid = lax.axis_index("c")
    sid = lax.axis_index("s")
    wid = cid * _NS + sid
    nch = E // _ECHUNK

    zh = pltpu.async_copy(z_hbm, a_chunk, zsem)

    def _zero16(k):
        deg_l[pl.ds(k * _L, _L)] = jnp.zeros((_L,), jnp.float32)
    _unrolled(N // _L, _zero16)

    p1bufs = [(cb0, wb0, sem0), (cb1, wb1, sem1)]

    def _issue1(ch):
        cb, wb, sem = p1bufs[ch % 2]
        sl = pl.ds(ch * _ECHUNK, _ECHUNK)
        return [pltpu.async_copy(col_hbm.at[sl], cb, sem),
                pltpu.async_copy(ew_hbm.at[sl], wb, sem)]

    pend1 = {0: _issue1(0)}
    for ch in range(nch):
        if ch + 1 < nch:
            pend1[(ch + 1) % 2] = _issue1(ch + 1)
        for h in pend1[ch % 2]:
            h.wait()
        cb, wb, _ = p1bufs[ch % 2]

        def _deg_step(k):
            c16 = cb[pl.ds(k * _L, _L)]
            w16 = wb[pl.ds(k * _L, _L)]
            plsc.addupdate_scatter(deg_l, [c16], w16)
        _unrolled(_ECHUNK // _L, _deg_step)

    def _dinv_step(r):
        dinv_l[pl.ds(r * _L, _L)] = _rsqrt16(1.0 + deg_l[pl.ds(r * _L, _L)])
    _unrolled(N // _L, _dinv_step)

    seq = [(p, ch) for p in range(_PASSES) for ch in range(nch)]
    bufs = [(cb0, wb0, rb0, sem0), (cb1, wb1, rb1, sem1)]

    def _issue(i):
        p, ch = seq[i]
        cb, wb, rb, sem = bufs[i % 2]
        sl = pl.ds(ch * _ECHUNK, _ECHUNK)
        return [pltpu.async_copy(col_hbm.at[sl], cb, sem),
                pltpu.async_copy(ew_hbm.at[sl], wb, sem),
                pltpu.async_copy(row_hbm.at[sl], rb, sem)]

    pending = {0: _issue(0)}
    for i, (p, ch) in enumerate(seq):
        base = (p * _NW + wid) * _ROWS_PER_TILE
        if ch == 0:
            zh.wait()
        if i + 1 < len(seq):
            pending[(i + 1) % 2] = _issue(i + 1)
        for h in pending[i % 2]:
            h.wait()
        cb, wb, rb, _ = bufs[i % 2]

        def _edge_step(k):
            c16 = cb[pl.ds(k * _L, _L)]
            r16 = rb[pl.ds(k * _L, _L)]
            w16 = wb[pl.ds(k * _L, _L)]
            m = (c16 >= base) & (c16 < base + _ROWS_PER_TILE)
            dr = plsc.load_gather(dinv_l, [r16])
            dc = plsc.load_gather(dinv_l, [c16])
            val = dr * w16 * dc
            idx = jnp.where(m, (c16 - base) * N + r16, 0)
            plsc.addupdate_scatter(a_chunk, [idx], val, mask=m)
        _unrolled(_ECHUNK // _L, _edge_step)

        if ch == nch - 1:
            for t in range(_ROWS_PER_TILE // _L):
                j16 = base + t * _L + lax.iota(jnp.int32, _L)
                d16 = plsc.load_gather(dinv_l, [j16])
                idx = (j16 - base) * N + j16
                plsc.addupdate_scatter(a_chunk, [idx], d16 * d16)
            pltpu.sync_copy(
                a_chunk, a_hbm.at[pl.ds(base * N, _ROWS_PER_TILE * N)])
            if p == 0:
                zh = pltpu.async_copy(z_hbm, a_chunk, zsem)


def _build_adj(edge_index, edge_weight):
    mesh = plsc.VectorSubcoreMesh(core_axis_name="c", subcore_axis_name="s")
    f = pl.kernel(
        _adj_body,
        out_type=jax.ShapeDtypeStruct((N * N,), jnp.float32),
        mesh=mesh,
        compiler_params=pltpu.CompilerParams(needs_layout_passes=False),
        scratch_types=[
            pltpu.VMEM((N,), jnp.float32),
            pltpu.VMEM((N,), jnp.float32),
            pltpu.VMEM((_ROWS_PER_TILE * N,), jnp.float32),
            pltpu.VMEM((_ECHUNK,), jnp.int32),
            pltpu.VMEM((_ECHUNK,), jnp.float32),
            pltpu.VMEM((_ECHUNK,), jnp.int32),
            pltpu.VMEM((_ECHUNK,), jnp.int32),
            pltpu.VMEM((_ECHUNK,), jnp.float32),
            pltpu.VMEM((_ECHUNK,), jnp.int32),
            pltpu.SemaphoreType.DMA,
            pltpu.SemaphoreType.DMA,
            pltpu.SemaphoreType.DMA,
        ],
    )
    row = edge_index[0]
    col = edge_index[1]
    zeros = jnp.zeros((_ROWS_PER_TILE * N,), jnp.float32)
    return f(row, col, edge_weight, zeros).reshape(N, N)


_NB = N // _RB
_NE = E // _EB


def _mega_body(adj_ref, x_ref, w_ref, b_ref, wq_ref, wqb_ref, emb_ref,
               w1_ref, w2_ref, wfb_ref, o_ref, h1_scr):
    i = pl.program_id(0)

    @pl.when(i < _NB)
    def _():
        t = lax.dot_general(
            adj_ref[:, :], x_ref[:, :], (((1,), (0,)), ((), ())))
        t = lax.dot_general(t, w_ref[:, :], (((1,), (1,)), ((), ())))
        h1_scr[pl.ds(i * _RB, _RB), :] = (
            jnp.tanh(t + b_ref[:, :]).astype(jnp.bfloat16))

    @pl.when(i >= _NB)
    def _():
        q = lax.dot_general(
            h1_scr[:, :], wq_ref[:, :].astype(jnp.bfloat16),
            (((1,), (1,)), ((), ())),
            preferred_element_type=jnp.float32)
        t = jnp.tanh(q + wqb_ref[0])
        f = jnp.sum(t * w1_ref[:, 0:1], axis=0, keepdims=True)
        g = lax.dot_general(
            w2_ref[:, :], emb_ref[:, :], (((1,), (1,)), ((), ())))
        o_ref[0] = f + g + wfb_ref[:, :]


def kernel(x, edge_index, edge_weight, gcn_W, gcn_b, Wq_W, Wq_b, emb, WF_W,
           WF_b):
    adj = _build_adj(edge_index, edge_weight)

    w1 = jnp.broadcast_to(WF_W[:, :N].reshape(N, 1), (N, 128))
    w2 = WF_W[:, N:]
    wqb = Wq_b.reshape(E // _EB, 1, _EB)

    def _e(i):
        return jnp.maximum(i - _NB, 0)

    out = pl.pallas_call(
        _mega_body,
        grid=(_NB + _NE,),
        in_specs=[
            pl.BlockSpec((_RB, N), lambda i: (jnp.minimum(i, _NB - 1), 0)),
            pl.BlockSpec((N, N), lambda i: (0, 0)),
            pl.BlockSpec((N, N), lambda i: (0, 0)),
            pl.BlockSpec((1, N), lambda i: (0, 0)),
            pl.BlockSpec((_EB, N), lambda i: (_e(i), 0)),
            pl.BlockSpec((1, 1, _EB), lambda i: (_e(i), 0, 0)),
            pl.BlockSpec((_EB, D), lambda i: (_e(i), 0)),
            pl.BlockSpec((N, 128), lambda i: (0, 0)),
            pl.BlockSpec((1, D), lambda i: (0, 0)),
            pl.BlockSpec((1, 1), lambda i: (0, 0)),
        ],
        out_specs=pl.BlockSpec((1, 1, _EB), lambda i: (_e(i), 0, 0)),
        out_shape=jax.ShapeDtypeStruct((E // _EB, 1, _EB), jnp.float32),
        scratch_shapes=[pltpu.VMEM((N, N), jnp.bfloat16)],
        compiler_params=pltpu.CompilerParams(
            vmem_limit_bytes=100 * 1024 * 1024),
    )(adj, x, gcn_W, gcn_b.reshape(1, N), Wq_W, wqb, emb, w1, w2,
      WF_b.reshape(1, 1))

    return out.reshape(E)


# speedup vs baseline: 1.0303x; 1.0303x over previous
import functools
import jax
import jax.numpy as jnp
from jax import lax
from jax.experimental import pallas as pl
from jax.experimental.pallas import tpu as pltpu
from jax.experimental.pallas import tpu_sc as plsc

N = 2048
E = 32768
D = 8

_RB = 256
_EB = 1024

_NC = 2
_NS = 16
_L = 16
_NW = _NC * _NS
_ROWS_PER_TILE = 32
_PASSES = N // (_NW * _ROWS_PER_TILE)
_ECHUNK = 4096


def _rsqrt16(x):
    i = plsc.bitcast(x, jnp.int32)
    y = plsc.bitcast(jnp.int32(0x5F3759DF) - (i >> 1), jnp.float32)
    for _ in range(3):
        y = y * (1.5 - 0.5 * x * y * y)
    return y


def _unrolled(n_vecs, body16, unroll=4):
    def _body(k, _):
        for u in range(unroll):
            body16(k * unroll + u)
        return 0
    lax.fori_loop(0, n_vecs // unroll, _body, 0)


def _adj_body(row_hbm, col_hbm, ew_hbm, z_hbm, a_hbm,
              deg_l, dinv_l, a_chunk,
              cb0, wb0, rb0, cb1, wb1, rb1,
              sem0, sem1, zsem):
    cid = lax.axis_index("c")
    sid = lax.axis_index("s")
    wid = cid * _NS + sid
    nch = E // _ECHUNK

    zh = pltpu.async_copy(z_hbm, a_chunk, zsem)

    def _zero16(k):
        deg_l[pl.ds(k * _L, _L)] = jnp.zeros((_L,), jnp.float32)
    _unrolled(N // _L, _zero16)

    p1bufs = [(cb0, wb0, sem0), (cb1, wb1, sem1)]

    def _issue1(ch):
        cb, wb, sem = p1bufs[ch % 2]
        sl = pl.ds(ch * _ECHUNK, _ECHUNK)
        return [pltpu.async_copy(col_hbm.at[sl], cb, sem),
                pltpu.async_copy(ew_hbm.at[sl], wb, sem)]

    pend1 = {0: _issue1(0)}
    for ch in range(nch):
        if ch + 1 < nch:
            pend1[(ch + 1) % 2] = _issue1(ch + 1)
        for h in pend1[ch % 2]:
            h.wait()
        cb, wb, _ = p1bufs[ch % 2]

        def _deg_step(k):
            c16 = cb[pl.ds(k * _L, _L)]
            w16 = wb[pl.ds(k * _L, _L)]
            plsc.addupdate_scatter(deg_l, [c16], w16)
        _unrolled(_ECHUNK // _L, _deg_step)

    def _dinv_step(r):
        dinv_l[pl.ds(r * _L, _L)] = _rsqrt16(1.0 + deg_l[pl.ds(r * _L, _L)])
    _unrolled(N // _L, _dinv_step)

    seq = [(p, ch) for p in range(_PASSES) for ch in range(nch)]
    bufs = [(cb0, wb0, rb0, sem0), (cb1, wb1, rb1, sem1)]

    def _issue(i):
        p, ch = seq[i]
        cb, wb, rb, sem = bufs[i % 2]
        sl = pl.ds(ch * _ECHUNK, _ECHUNK)
        return [pltpu.async_copy(col_hbm.at[sl], cb, sem),
                pltpu.async_copy(ew_hbm.at[sl], wb, sem),
                pltpu.async_copy(row_hbm.at[sl], rb, sem)]

    pending = {0: _issue(0)}
    for i, (p, ch) in enumerate(seq):
        base = (p * _NW + wid) * _ROWS_PER_TILE
        if ch == 0:
            zh.wait()
        if i + 1 < len(seq):
            pending[(i + 1) % 2] = _issue(i + 1)
        for h in pending[i % 2]:
            h.wait()
        cb, wb, rb, _ = bufs[i % 2]

        def _edge_step(k):
            c16 = cb[pl.ds(k * _L, _L)]
            r16 = rb[pl.ds(k * _L, _L)]
            w16 = wb[pl.ds(k * _L, _L)]
            m = (c16 >= base) & (c16 < base + _ROWS_PER_TILE)
            dr = plsc.load_gather(dinv_l, [r16])
            dc = plsc.load_gather(dinv_l, [c16])
            val = dr * w16 * dc
            idx = jnp.where(m, (c16 - base) * N + r16, 0)
            plsc.addupdate_scatter(a_chunk, [idx], val, mask=m)
        _unrolled(_ECHUNK // _L, _edge_step)

        if ch == nch - 1:
            for t in range(_ROWS_PER_TILE // _L):
                j16 = base + t * _L + lax.iota(jnp.int32, _L)
                d16 = plsc.load_gather(dinv_l, [j16])
                idx = (j16 - base) * N + j16
                plsc.addupdate_scatter(a_chunk, [idx], d16 * d16)
            pltpu.sync_copy(
                a_chunk, a_hbm.at[pl.ds(base * N, _ROWS_PER_TILE * N)])
            if p == 0:
                zh = pltpu.async_copy(z_hbm, a_chunk, zsem)


def _build_adj(edge_index, edge_weight):
    mesh = plsc.VectorSubcoreMesh(core_axis_name="c", subcore_axis_name="s")
    f = pl.kernel(
        _adj_body,
        out_type=jax.ShapeDtypeStruct((N * N,), jnp.float32),
        mesh=mesh,
        compiler_params=pltpu.CompilerParams(needs_layout_passes=False),
        scratch_types=[
            pltpu.VMEM((N,), jnp.float32),
            pltpu.VMEM((N,), jnp.float32),
            pltpu.VMEM((_ROWS_PER_TILE * N,), jnp.float32),
            pltpu.VMEM((_ECHUNK,), jnp.int32),
            pltpu.VMEM((_ECHUNK,), jnp.float32),
            pltpu.VMEM((_ECHUNK,), jnp.int32),
            pltpu.VMEM((_ECHUNK,), jnp.int32),
            pltpu.VMEM((_ECHUNK,), jnp.float32),
            pltpu.VMEM((_ECHUNK,), jnp.int32),
            pltpu.SemaphoreType.DMA,
            pltpu.SemaphoreType.DMA,
            pltpu.SemaphoreType.DMA,
        ],
    )
    row = edge_index[0]
    col = edge_index[1]
    zeros = jnp.zeros((_ROWS_PER_TILE * N,), jnp.float32)
    return f(row, col, edge_weight, zeros).reshape(N, N)


def _h1_body(a_ref, x_ref, w_ref, b_ref, o_ref):
    t = lax.dot_general(a_ref[:, :], x_ref[:, :], (((1,), (0,)), ((), ())))
    t = lax.dot_general(t, w_ref[:, :], (((1,), (1,)), ((), ())))
    o_ref[:, :] = jnp.tanh(t + b_ref[:, :]).astype(jnp.bfloat16)


def _fused_body(h1_ref, wq_ref, wqb_ref, emb_ref, w1_ref, w2_ref, wfb_ref,
                o_ref):
    q = lax.dot_general(
        h1_ref[:, :], wq_ref[:, :].astype(jnp.bfloat16),
        (((1,), (1,)), ((), ())),
        preferred_element_type=jnp.float32)
    t = jnp.tanh(q + wqb_ref[0])
    f = jnp.sum(t * w1_ref[:, 0:1], axis=0, keepdims=True)
    g = lax.dot_general(
        w2_ref[:, :], emb_ref[:, :], (((1,), (1,)), ((), ())))
    o_ref[0] = f + g + wfb_ref[:, :]


def kernel(x, edge_index, edge_weight, gcn_W, gcn_b, Wq_W, Wq_b, emb, WF_W,
           WF_b):
    adj = _build_adj(edge_index, edge_weight)

    w1 = jnp.broadcast_to(WF_W[:, :N].reshape(N, 1), (N, 128))
    w2 = WF_W[:, N:]
    wqb = Wq_b.reshape(E // _EB, 1, _EB)

    h1 = pl.pallas_call(
        _h1_body,
        grid=(N // _RB,),
        in_specs=[
            pl.BlockSpec((_RB, N), lambda i: (i, 0)),
            pl.BlockSpec((N, N), lambda i: (0, 0)),
            pl.BlockSpec((N, N), lambda i: (0, 0)),
            pl.BlockSpec((1, N), lambda i: (0, 0)),
        ],
        out_specs=pl.BlockSpec((_RB, N), lambda i: (i, 0)),
        out_shape=jax.ShapeDtypeStruct((N, N), jnp.bfloat16),
    )(adj, x, gcn_W, gcn_b.reshape(1, N))

    out = pl.pallas_call(
        _fused_body,
        grid=(E // _EB,),
        in_specs=[
            pl.BlockSpec((N, N), lambda i: (0, 0)),
            pl.BlockSpec((_EB, N), lambda i: (i, 0)),
            pl.BlockSpec((1, 1, _EB), lambda i: (i, 0, 0)),
            pl.BlockSpec((_EB, D), lambda i: (i, 0)),
            pl.BlockSpec((N, 128), lambda i: (0, 0)),
            pl.BlockSpec((1, D), lambda i: (0, 0)),
            pl.BlockSpec((1, 1), lambda i: (0, 0)),
        ],
        out_specs=pl.BlockSpec((1, 1, _EB), lambda i: (i, 0, 0)),
        out_shape=jax.ShapeDtypeStruct((E // _EB, 1, _EB), jnp.float32),
    )(h1, Wq_W, wqb, emb, w1, w2, WF_b.reshape(1, 1))

    return out.reshape(E)


# speedup vs baseline: 1.0377x; 1.0072x over previous
import functools
import jax
import jax.numpy as jnp
from jax import lax
from jax.experimental import pallas as pl
from jax.experimental.pallas import tpu as pltpu
from jax.experimental.pallas import tpu_sc as plsc

N = 2048
E = 32768
D = 8

_RB = 256
_EB = 1024

_NC = 2
_NS = 16
_L = 16
_NW = _NC * _NS
_ROWS_PER_TILE = 32
_PASSES = N // (_NW * _ROWS_PER_TILE)
_ECHUNK = 8192


def _rsqrt16(x):
    i = plsc.bitcast(x, jnp.int32)
    y = plsc.bitcast(jnp.int32(0x5F3759DF) - (i >> 1), jnp.float32)
    for _ in range(3):
        y = y * (1.5 - 0.5 * x * y * y)
    return y


def _unrolled(n_vecs, body16, unroll=4):
    def _body(k, _):
        for u in range(unroll):
            body16(k * unroll + u)
        return 0
    lax.fori_loop(0, n_vecs // unroll, _body, 0)


def _adj_body(row_hbm, col_hbm, ew_hbm, a_hbm,
              deg_l, dinv_l, a_chunk,
              cb0, wb0, rb0, cb1, wb1, rb1,
              sem0, sem1):
    cid = lax.axis_index("c")
    sid = lax.axis_index("s")
    wid = cid * _NS + sid
    nch = E // _ECHUNK

    def _zero16(k):
        deg_l[pl.ds(k * _L, _L)] = jnp.zeros((_L,), jnp.float32)
    _unrolled(N // _L, _zero16)

    p1bufs = [(cb0, wb0, sem0), (cb1, wb1, sem1)]

    def _issue1(ch):
        cb, wb, sem = p1bufs[ch % 2]
        sl = pl.ds(ch * _ECHUNK, _ECHUNK)
        return [pltpu.async_copy(col_hbm.at[sl], cb, sem),
                pltpu.async_copy(ew_hbm.at[sl], wb, sem)]

    pend1 = {0: _issue1(0)}
    for ch in range(nch):
        if ch + 1 < nch:
            pend1[(ch + 1) % 2] = _issue1(ch + 1)
        for h in pend1[ch % 2]:
            h.wait()
        cb, wb, _ = p1bufs[ch % 2]

        def _deg_step(k):
            c16 = cb[pl.ds(k * _L, _L)]
            w16 = wb[pl.ds(k * _L, _L)]
            plsc.addupdate_scatter(deg_l, [c16], w16)
        _unrolled(_ECHUNK // _L, _deg_step)

    def _dinv_step(r):
        dinv_l[pl.ds(r * _L, _L)] = _rsqrt16(1.0 + deg_l[pl.ds(r * _L, _L)])
    _unrolled(N // _L, _dinv_step)

    seq = [(p, ch) for p in range(_PASSES) for ch in range(nch)]
    bufs = [(cb0, wb0, rb0, sem0), (cb1, wb1, rb1, sem1)]

    def _issue(i):
        p, ch = seq[i]
        cb, wb, rb, sem = bufs[i % 2]
        sl = pl.ds(ch * _ECHUNK, _ECHUNK)
        return [pltpu.async_copy(col_hbm.at[sl], cb, sem),
                pltpu.async_copy(ew_hbm.at[sl], wb, sem),
                pltpu.async_copy(row_hbm.at[sl], rb, sem)]

    pending = {0: _issue(0)}
    for i, (p, ch) in enumerate(seq):
        base = (p * _NW + wid) * _ROWS_PER_TILE
        if ch == 0:
            def _zchunk(k):
                a_chunk[pl.ds(k * _L, _L)] = jnp.zeros((_L,), jnp.float32)
            _unrolled((_ROWS_PER_TILE * N) // _L, _zchunk, unroll=8)
        if i + 1 < len(seq):
            pending[(i + 1) % 2] = _issue(i + 1)
        for h in pending[i % 2]:
            h.wait()
        cb, wb, rb, _ = bufs[i % 2]

        def _edge_step(k):
            c16 = cb[pl.ds(k * _L, _L)]
            r16 = rb[pl.ds(k * _L, _L)]
            w16 = wb[pl.ds(k * _L, _L)]
            m = (c16 >= base) & (c16 < base + _ROWS_PER_TILE)
            dr = plsc.load_gather(dinv_l, [r16])
            dc = plsc.load_gather(dinv_l, [c16])
            val = dr * w16 * dc
            idx = jnp.where(m, (c16 - base) * N + r16, 0)
            plsc.addupdate_scatter(a_chunk, [idx], val, mask=m)
        _unrolled(_ECHUNK // _L, _edge_step)

        if ch == nch - 1:
            for t in range(_ROWS_PER_TILE // _L):
                j16 = base + t * _L + lax.iota(jnp.int32, _L)
                d16 = plsc.load_gather(dinv_l, [j16])
                idx = (j16 - base) * N + j16
                plsc.addupdate_scatter(a_chunk, [idx], d16 * d16)
            pltpu.sync_copy(
                a_chunk, a_hbm.at[pl.ds(base * N, _ROWS_PER_TILE * N)])


def _build_adj(edge_index, edge_weight):
    mesh = plsc.VectorSubcoreMesh(core_axis_name="c", subcore_axis_name="s")
    f = pl.kernel(
        _adj_body,
        out_type=jax.ShapeDtypeStruct((N * N,), jnp.float32),
        mesh=mesh,
        compiler_params=pltpu.CompilerParams(needs_layout_passes=False),
        scratch_types=[
            pltpu.VMEM((N,), jnp.float32),
            pltpu.VMEM((N,), jnp.float32),
            pltpu.VMEM((_ROWS_PER_TILE * N,), jnp.float32),
            pltpu.VMEM((_ECHUNK,), jnp.int32),
            pltpu.VMEM((_ECHUNK,), jnp.float32),
            pltpu.VMEM((_ECHUNK,), jnp.int32),
            pltpu.VMEM((_ECHUNK,), jnp.int32),
            pltpu.VMEM((_ECHUNK,), jnp.float32),
            pltpu.VMEM((_ECHUNK,), jnp.int32),
            pltpu.SemaphoreType.DMA,
            pltpu.SemaphoreType.DMA,
        ],
    )
    row = edge_index[0]
    col = edge_index[1]
    return f(row, col, edge_weight).reshape(N, N)


def _h1_body(a_ref, x_ref, w_ref, b_ref, o_ref):
    t = lax.dot_general(a_ref[:, :], x_ref[:, :], (((1,), (0,)), ((), ())))
    t = lax.dot_general(t, w_ref[:, :], (((1,), (1,)), ((), ())))
    o_ref[:, :] = jnp.tanh(t + b_ref[:, :]).astype(jnp.bfloat16)


def _fused_body(h1_ref, wq_ref, wqb_ref, emb_ref, w1_ref, w2_ref, wfb_ref,
                o_ref):
    q = lax.dot_general(
        h1_ref[:, :], wq_ref[:, :].astype(jnp.bfloat16),
        (((1,), (1,)), ((), ())),
        preferred_element_type=jnp.float32)
    t = jnp.tanh(q + wqb_ref[0])
    f = jnp.sum(t * w1_ref[:, 0:1], axis=0, keepdims=True)
    g = lax.dot_general(
        w2_ref[:, :], emb_ref[:, :], (((1,), (1,)), ((), ())))
    o_ref[0] = f + g + wfb_ref[:, :]


def kernel(x, edge_index, edge_weight, gcn_W, gcn_b, Wq_W, Wq_b, emb, WF_W,
           WF_b):
    adj = _build_adj(edge_index, edge_weight)

    w1 = jnp.broadcast_to(WF_W[:, :N].reshape(N, 1), (N, 128))
    w2 = WF_W[:, N:]
    wqb = Wq_b.reshape(E // _EB, 1, _EB)

    h1 = pl.pallas_call(
        _h1_body,
        grid=(N // _RB,),
        in_specs=[
            pl.BlockSpec((_RB, N), lambda i: (i, 0)),
            pl.BlockSpec((N, N), lambda i: (0, 0)),
            pl.BlockSpec((N, N), lambda i: (0, 0)),
            pl.BlockSpec((1, N), lambda i: (0, 0)),
        ],
        out_specs=pl.BlockSpec((_RB, N), lambda i: (i, 0)),
        out_shape=jax.ShapeDtypeStruct((N, N), jnp.bfloat16),
    )(adj, x, gcn_W, gcn_b.reshape(1, N))

    out = pl.pallas_call(
        _fused_body,
        grid=(E // _EB,),
        in_specs=[
            pl.BlockSpec((N, N), lambda i: (0, 0)),
            pl.BlockSpec((_EB, N), lambda i: (i, 0)),
            pl.BlockSpec((1, 1, _EB), lambda i: (i, 0, 0)),
            pl.BlockSpec((_EB, D), lambda i: (i, 0)),
            pl.BlockSpec((N, 128), lambda i: (0, 0)),
            pl.BlockSpec((1, D), lambda i: (0, 0)),
            pl.BlockSpec((1, 1), lambda i: (0, 0)),
        ],
        out_specs=pl.BlockSpec((1, 1, _EB), lambda i: (i, 0, 0)),
        out_shape=jax.ShapeDtypeStruct((E // _EB, 1, _EB), jnp.float32),
    )(h1, Wq_W, wqb, emb, w1, w2, WF_b.reshape(1, 1))

    return out.reshape(E)


# speedup vs baseline: 1.0408x; 1.0030x over previous
import functools
import jax
import jax.numpy as jnp
from jax import lax
from jax.experimental import pallas as pl
from jax.experimental.pallas import tpu as pltpu
from jax.experimental.pallas import tpu_sc as plsc

N = 2048
E = 32768
D = 8

_RB = 256
_EB = 1024

_NC = 2
_NS = 16
_L = 16
_NW = _NC * _NS
_ROWS_PER_TILE = 32
_PASSES = N // (_NW * _ROWS_PER_TILE)
_ECHUNK = 8192


def _rsqrt16(x):
    i = plsc.bitcast(x, jnp.int32)
    y = plsc.bitcast(jnp.int32(0x5F3759DF) - (i >> 1), jnp.float32)
    for _ in range(3):
        y = y * (1.5 - 0.5 * x * y * y)
    return y


def _unrolled(n_vecs, body16, unroll=4):
    def _body(k, _):
        for u in range(unroll):
            body16(k * unroll + u)
        return 0
    lax.fori_loop(0, n_vecs // unroll, _body, 0)


def _adj_body(ei_hbm, ew_hbm, a_hbm,
              deg_l, dinv_l, a_chunk,
              cb0, wb0, rb0, cb1, wb1, rb1,
              sem0, sem1):
    cid = lax.axis_index("c")
    sid = lax.axis_index("s")
    wid = cid * _NS + sid
    nch = E // _ECHUNK

    def _zero16(k):
        deg_l[pl.ds(k * _L, _L)] = jnp.zeros((_L,), jnp.float32)
    _unrolled(N // _L, _zero16)

    p1bufs = [(cb0, wb0, sem0), (cb1, wb1, sem1)]

    def _issue1(ch):
        cb, wb, sem = p1bufs[ch % 2]
        sl = pl.ds(ch * _ECHUNK, _ECHUNK)
        return [pltpu.async_copy(ei_hbm.at[1, sl], cb, sem),
                pltpu.async_copy(ew_hbm.at[sl], wb, sem)]

    pend1 = {0: _issue1(0)}
    for ch in range(nch):
        if ch + 1 < nch:
            pend1[(ch + 1) % 2] = _issue1(ch + 1)
        for h in pend1[ch % 2]:
            h.wait()
        cb, wb, _ = p1bufs[ch % 2]

        def _deg_step(k):
            c16 = cb[pl.ds(k * _L, _L)]
            w16 = wb[pl.ds(k * _L, _L)]
            plsc.addupdate_scatter(deg_l, [c16], w16)
        _unrolled(_ECHUNK // _L, _deg_step)

    def _dinv_step(r):
        dinv_l[pl.ds(r * _L, _L)] = _rsqrt16(1.0 + deg_l[pl.ds(r * _L, _L)])
    _unrolled(N // _L, _dinv_step)

    seq = [(p, ch) for p in range(_PASSES) for ch in range(nch)]
    bufs = [(cb0, wb0, rb0, sem0), (cb1, wb1, rb1, sem1)]

    def _issue(i):
        p, ch = seq[i]
        cb, wb, rb, sem = bufs[i % 2]
        sl = pl.ds(ch * _ECHUNK, _ECHUNK)
        return [pltpu.async_copy(ei_hbm.at[1, sl], cb, sem),
                pltpu.async_copy(ew_hbm.at[sl], wb, sem),
                pltpu.async_copy(ei_hbm.at[0, sl], rb, sem)]

    pending = {0: _issue(0)}
    for i, (p, ch) in enumerate(seq):
        base = (p * _NW + wid) * _ROWS_PER_TILE
        if ch == 0:
            def _zchunk(k):
                a_chunk[pl.ds(k * _L, _L)] = jnp.zeros((_L,), jnp.float32)
            _unrolled((_ROWS_PER_TILE * N) // _L, _zchunk, unroll=8)
        if i + 1 < len(seq):
            pending[(i + 1) % 2] = _issue(i + 1)
        for h in pending[i % 2]:
            h.wait()
        cb, wb, rb, _ = bufs[i % 2]

        def _edge_step(k):
            c16 = cb[pl.ds(k * _L, _L)]
            r16 = rb[pl.ds(k * _L, _L)]
            w16 = wb[pl.ds(k * _L, _L)]
            m = (c16 >= base) & (c16 < base + _ROWS_PER_TILE)
            dr = plsc.load_gather(dinv_l, [r16])
            dc = plsc.load_gather(dinv_l, [c16])
            val = dr * w16 * dc
            idx = jnp.where(m, (c16 - base) * N + r16, 0)
            plsc.addupdate_scatter(a_chunk, [idx], val, mask=m)
        _unrolled(_ECHUNK // _L, _edge_step)

        if ch == nch - 1:
            for t in range(_ROWS_PER_TILE // _L):
                j16 = base + t * _L + lax.iota(jnp.int32, _L)
                d16 = plsc.load_gather(dinv_l, [j16])
                idx = (j16 - base) * N + j16
                plsc.addupdate_scatter(a_chunk, [idx], d16 * d16)
            pltpu.sync_copy(
                a_chunk, a_hbm.at[pl.ds(base * N, _ROWS_PER_TILE * N)])


def _build_adj(edge_index, edge_weight):
    mesh = plsc.VectorSubcoreMesh(core_axis_name="c", subcore_axis_name="s")
    f = pl.kernel(
        _adj_body,
        out_type=jax.ShapeDtypeStruct((N * N,), jnp.float32),
        mesh=mesh,
        compiler_params=pltpu.CompilerParams(needs_layout_passes=False),
        scratch_types=[
            pltpu.VMEM((N,), jnp.float32),
            pltpu.VMEM((N,), jnp.float32),
            pltpu.VMEM((_ROWS_PER_TILE * N,), jnp.float32),
            pltpu.VMEM((_ECHUNK,), jnp.int32),
            pltpu.VMEM((_ECHUNK,), jnp.float32),
            pltpu.VMEM((_ECHUNK,), jnp.int32),
            pltpu.VMEM((_ECHUNK,), jnp.int32),
            pltpu.VMEM((_ECHUNK,), jnp.float32),
            pltpu.VMEM((_ECHUNK,), jnp.int32),
            pltpu.SemaphoreType.DMA,
            pltpu.SemaphoreType.DMA,
        ],
    )
    return f(edge_index, edge_weight).reshape(N, N)


def _h1_body(a_ref, x_ref, w_ref, b_ref, o_ref):
    t = lax.dot_general(a_ref[:, :], x_ref[:, :], (((1,), (0,)), ((), ())))
    t = lax.dot_general(t, w_ref[:, :], (((1,), (1,)), ((), ())))
    o_ref[:, :] = jnp.tanh(t + b_ref[:, :]).astype(jnp.bfloat16)


def _fused_body(h1_ref, wq_ref, wqb_ref, emb_ref, w1_ref, w2_ref, wfb_ref,
                o_ref):
    q = lax.dot_general(
        h1_ref[:, :], wq_ref[:, :].astype(jnp.bfloat16),
        (((1,), (1,)), ((), ())),
        preferred_element_type=jnp.float32)
    t = jnp.tanh(q + wqb_ref[0])
    f = jnp.sum(t * w1_ref[:, 0:1], axis=0, keepdims=True)
    g = lax.dot_general(
        w2_ref[:, :], emb_ref[:, :], (((1,), (1,)), ((), ())))
    o_ref[0] = f + g + wfb_ref[:, :]


def kernel(x, edge_index, edge_weight, gcn_W, gcn_b, Wq_W, Wq_b, emb, WF_W,
           WF_b):
    adj = _build_adj(edge_index, edge_weight)

    w1 = jnp.broadcast_to(WF_W[:, :N].reshape(N, 1), (N, 8))
    w2 = WF_W[:, N:]
    wqb = Wq_b.reshape(E // _EB, 1, _EB)

    h1 = pl.pallas_call(
        _h1_body,
        grid=(N // _RB,),
        in_specs=[
            pl.BlockSpec((_RB, N), lambda i: (i, 0)),
            pl.BlockSpec((N, N), lambda i: (0, 0)),
            pl.BlockSpec((N, N), lambda i: (0, 0)),
            pl.BlockSpec((1, N), lambda i: (0, 0)),
        ],
        out_specs=pl.BlockSpec((_RB, N), lambda i: (i, 0)),
        out_shape=jax.ShapeDtypeStruct((N, N), jnp.bfloat16),
    )(adj, x, gcn_W, gcn_b.reshape(1, N))

    out = pl.pallas_call(
        _fused_body,
        grid=(E // _EB,),
        in_specs=[
            pl.BlockSpec((N, N), lambda i: (0, 0)),
            pl.BlockSpec((_EB, N), lambda i: (i, 0)),
            pl.BlockSpec((1, 1, _EB), lambda i: (i, 0, 0)),
            pl.BlockSpec((_EB, D), lambda i: (i, 0)),
            pl.BlockSpec((N, 8), lambda i: (0, 0)),
            pl.BlockSpec((1, D), lambda i: (0, 0)),
            pl.BlockSpec((1, 1), lambda i: (0, 0)),
        ],
        out_specs=pl.BlockSpec((1, 1, _EB), lambda i: (i, 0, 0)),
        out_shape=jax.ShapeDtypeStruct((E // _EB, 1, _EB), jnp.float32),
    )(h1, Wq_W, wqb, emb, w1, w2, WF_b.reshape(1, 1))

    return out.reshape(E)


# speedup vs baseline: 1.0418x; 1.0010x over previous
import functools
import jax
import jax.numpy as jnp
from jax import lax
from jax.experimental import pallas as pl
from jax.experimental.pallas import tpu as pltpu
from jax.experimental.pallas import tpu_sc as plsc

N = 2048
E = 32768
D = 8

_RB = 256
_EB = 2048

_NC = 2
_NS = 16
_L = 16
_NW = _NC * _NS
_ROWS_PER_TILE = 32
_PASSES = N // (_NW * _ROWS_PER_TILE)
_ECHUNK = 8192


def _rsqrt16(x):
    i = plsc.bitcast(x, jnp.int32)
    y = plsc.bitcast(jnp.int32(0x5F3759DF) - (i >> 1), jnp.float32)
    for _ in range(3):
        y = y * (1.5 - 0.5 * x * y * y)
    return y


def _unrolled(n_vecs, body16, unroll=4):
    def _body(k, _):
        for u in range(unroll):
            body16(k * unroll + u)
        return 0
    lax.fori_loop(0, n_vecs // unroll, _body, 0)


def _adj_body(ei_hbm, ew_hbm, a_hbm,
              deg_l, dinv_l, a_chunk,
              cb0, wb0, rb0, cb1, wb1, rb1,
              sem0, sem1):
    cid = lax.axis_index("c")
    sid = lax.axis_index("s")
    wid = cid * _NS + sid
    nch = E // _ECHUNK

    def _zero16(k):
        deg_l[pl.ds(k * _L, _L)] = jnp.zeros((_L,), jnp.float32)
    _unrolled(N // _L, _zero16)

    p1bufs = [(cb0, wb0, sem0), (cb1, wb1, sem1)]

    def _issue1(ch):
        cb, wb, sem = p1bufs[ch % 2]
        sl = pl.ds(ch * _ECHUNK, _ECHUNK)
        return [pltpu.async_copy(ei_hbm.at[1, sl], cb, sem),
                pltpu.async_copy(ew_hbm.at[sl], wb, sem)]

    pend1 = {0: _issue1(0)}
    for ch in range(nch):
        if ch + 1 < nch:
            pend1[(ch + 1) % 2] = _issue1(ch + 1)
        for h in pend1[ch % 2]:
            h.wait()
        cb, wb, _ = p1bufs[ch % 2]

        def _deg_step(k):
            c16 = cb[pl.ds(k * _L, _L)]
            w16 = wb[pl.ds(k * _L, _L)]
            plsc.addupdate_scatter(deg_l, [c16], w16)
        _unrolled(_ECHUNK // _L, _deg_step)

    def _dinv_step(r):
        dinv_l[pl.ds(r * _L, _L)] = _rsqrt16(1.0 + deg_l[pl.ds(r * _L, _L)])
    _unrolled(N // _L, _dinv_step)

    seq = [(p, ch) for p in range(_PASSES) for ch in range(nch)]
    bufs = [(cb0, wb0, rb0, sem0), (cb1, wb1, rb1, sem1)]

    def _issue(i):
        p, ch = seq[i]
        cb, wb, rb, sem = bufs[i % 2]
        sl = pl.ds(ch * _ECHUNK, _ECHUNK)
        return [pltpu.async_copy(ei_hbm.at[1, sl], cb, sem),
                pltpu.async_copy(ew_hbm.at[sl], wb, sem),
                pltpu.async_copy(ei_hbm.at[0, sl], rb, sem)]

    pending = {0: _issue(0)}
    for i, (p, ch) in enumerate(seq):
        base = (p * _NW + wid) * _ROWS_PER_TILE
        if ch == 0:
            def _zchunk(k):
                a_chunk[pl.ds(k * _L, _L)] = jnp.zeros((_L,), jnp.float32)
            _unrolled((_ROWS_PER_TILE * N) // _L, _zchunk, unroll=8)
        if i + 1 < len(seq):
            pending[(i + 1) % 2] = _issue(i + 1)
        for h in pending[i % 2]:
            h.wait()
        cb, wb, rb, _ = bufs[i % 2]

        def _edge_step(k):
            c16 = cb[pl.ds(k * _L, _L)]
            r16 = rb[pl.ds(k * _L, _L)]
            w16 = wb[pl.ds(k * _L, _L)]
            m = (c16 >= base) & (c16 < base + _ROWS_PER_TILE)
            dr = plsc.load_gather(dinv_l, [r16])
            dc = plsc.load_gather(dinv_l, [c16])
            val = dr * w16 * dc
            idx = jnp.where(m, (c16 - base) * N + r16, 0)
            plsc.addupdate_scatter(a_chunk, [idx], val, mask=m)
        _unrolled(_ECHUNK // _L, _edge_step, unroll=8)

        if ch == nch - 1:
            for t in range(_ROWS_PER_TILE // _L):
                j16 = base + t * _L + lax.iota(jnp.int32, _L)
                d16 = plsc.load_gather(dinv_l, [j16])
                idx = (j16 - base) * N + j16
                plsc.addupdate_scatter(a_chunk, [idx], d16 * d16)
            pltpu.sync_copy(
                a_chunk, a_hbm.at[pl.ds(base * N, _ROWS_PER_TILE * N)])


def _build_adj(edge_index, edge_weight):
    mesh = plsc.VectorSubcoreMesh(core_axis_name="c", subcore_axis_name="s")
    f = pl.kernel(
        _adj_body,
        out_type=jax.ShapeDtypeStruct((N * N,), jnp.float32),
        mesh=mesh,
        compiler_params=pltpu.CompilerParams(needs_layout_passes=False),
        scratch_types=[
            pltpu.VMEM((N,), jnp.float32),
            pltpu.VMEM((N,), jnp.float32),
            pltpu.VMEM((_ROWS_PER_TILE * N,), jnp.float32),
            pltpu.VMEM((_ECHUNK,), jnp.int32),
            pltpu.VMEM((_ECHUNK,), jnp.float32),
            pltpu.VMEM((_ECHUNK,), jnp.int32),
            pltpu.VMEM((_ECHUNK,), jnp.int32),
            pltpu.VMEM((_ECHUNK,), jnp.float32),
            pltpu.VMEM((_ECHUNK,), jnp.int32),
            pltpu.SemaphoreType.DMA,
            pltpu.SemaphoreType.DMA,
        ],
    )
    return f(edge_index, edge_weight).reshape(N, N)


def _h1_body(a_ref, x_ref, w_ref, b_ref, o_ref):
    t = lax.dot_general(a_ref[:, :], x_ref[:, :], (((1,), (0,)), ((), ())))
    t = lax.dot_general(t, w_ref[:, :], (((1,), (1,)), ((), ())))
    o_ref[:, :] = jnp.tanh(t + b_ref[:, :]).astype(jnp.bfloat16)


def _fused_body(h1_ref, wq_ref, wqb_ref, emb_ref, w1_ref, w2_ref, wfb_ref,
                o_ref):
    q = lax.dot_general(
        h1_ref[:, :], wq_ref[:, :].astype(jnp.bfloat16),
        (((1,), (1,)), ((), ())),
        preferred_element_type=jnp.float32)
    t = jnp.tanh(q + wqb_ref[0])
    f = jnp.sum(t * w1_ref[:, 0:1], axis=0, keepdims=True)
    g = lax.dot_general(
        w2_ref[:, :], emb_ref[:, :], (((1,), (1,)), ((), ())))
    o_ref[0] = f + g + wfb_ref[:, :]


def kernel(x, edge_index, edge_weight, gcn_W, gcn_b, Wq_W, Wq_b, emb, WF_W,
           WF_b):
    adj = _build_adj(edge_index, edge_weight)

    w1 = jnp.broadcast_to(WF_W[:, :N].reshape(N, 1), (N, 8))
    w2 = WF_W[:, N:]
    wqb = Wq_b.reshape(E // _EB, 1, _EB)

    h1 = pl.pallas_call(
        _h1_body,
        grid=(N // _RB,),
        in_specs=[
            pl.BlockSpec((_RB, N), lambda i: (i, 0)),
            pl.BlockSpec((N, N), lambda i: (0, 0)),
            pl.BlockSpec((N, N), lambda i: (0, 0)),
            pl.BlockSpec((1, N), lambda i: (0, 0)),
        ],
        out_specs=pl.BlockSpec((_RB, N), lambda i: (i, 0)),
        out_shape=jax.ShapeDtypeStruct((N, N), jnp.bfloat16),
    )(adj, x, gcn_W, gcn_b.reshape(1, N))

    out = pl.pallas_call(
        _fused_body,
        grid=(E // _EB,),
        in_specs=[
            pl.BlockSpec((N, N), lambda i: (0, 0)),
            pl.BlockSpec((_EB, N), lambda i: (i, 0)),
            pl.BlockSpec((1, 1, _EB), lambda i: (i, 0, 0)),
            pl.BlockSpec((_EB, D), lambda i: (i, 0)),
            pl.BlockSpec((N, 8), lambda i: (0, 0)),
            pl.BlockSpec((1, D), lambda i: (0, 0)),
            pl.BlockSpec((1, 1), lambda i: (0, 0)),
        ],
        out_specs=pl.BlockSpec((1, 1, _EB), lambda i: (i, 0, 0)),
        out_shape=jax.ShapeDtypeStruct((E // _EB, 1, _EB), jnp.float32),
        compiler_params=pltpu.CompilerParams(
            vmem_limit_bytes=63 * 1024 * 1024),
    )(h1, Wq_W, wqb, emb, w1, w2, WF_b.reshape(1, 1))

    return out.reshape(E)


# speedup vs baseline: 1.0424x; 1.0005x over previous
import jax
import jax.numpy as jnp
from jax import lax
from jax.experimental import pallas as pl
from jax.experimental.pallas import tpu as pltpu
from jax.experimental.pallas import tpu_sc as plsc

N = 2048
E = 32768
D = 8

_RB = 256
_EB = 2048

_NC = 2
_NS = 16
_L = 16
_NW = _NC * _NS
_ROWS_PER_TILE = 32
_PASSES = N // (_NW * _ROWS_PER_TILE)
_ECHUNK = 8192


def _rsqrt16(x):
    i = plsc.bitcast(x, jnp.int32)
    y = plsc.bitcast(jnp.int32(0x5F3759DF) - (i >> 1), jnp.float32)
    for _ in range(3):
        y = y * (1.5 - 0.5 * x * y * y)
    return y


def _unrolled(n_vecs, body16, unroll=4):
    def _body(k, _):
        for u in range(unroll):
            body16(k * unroll + u)
        return 0
    lax.fori_loop(0, n_vecs // unroll, _body, 0)


def _adj_body(ei_hbm, ew_hbm, a_hbm,
              deg_l, dinv_l, a_chunk,
              cb0, wb0, rb0, cb1, wb1, rb1,
              sem0, sem1):
    cid = lax.axis_index("c")
    sid = lax.axis_index("s")
    wid = cid * _NS + sid
    nch = E // _ECHUNK

    def _zero16(k):
        deg_l[pl.ds(k * _L, _L)] = jnp.zeros((_L,), jnp.float32)
    _unrolled(N // _L, _zero16)

    p1bufs = [(cb0, wb0, sem0), (cb1, wb1, sem1)]

    def _issue1(ch):
        cb, wb, sem = p1bufs[ch % 2]
        sl = pl.ds(ch * _ECHUNK, _ECHUNK)
        return [pltpu.async_copy(ei_hbm.at[1, sl], cb, sem),
                pltpu.async_copy(ew_hbm.at[sl], wb, sem)]

    pend1 = {0: _issue1(0)}
    for ch in range(nch):
        if ch + 1 < nch:
            pend1[(ch + 1) % 2] = _issue1(ch + 1)
        for h in pend1[ch % 2]:
            h.wait()
        cb, wb, _ = p1bufs[ch % 2]

        def _deg_step(k):
            c16 = cb[pl.ds(k * _L, _L)]
            w16 = wb[pl.ds(k * _L, _L)]
            plsc.addupdate_scatter(deg_l, [c16], w16)
        _unrolled(_ECHUNK // _L, _deg_step)

    def _dinv_step(r):
        dinv_l[pl.ds(r * _L, _L)] = _rsqrt16(1.0 + deg_l[pl.ds(r * _L, _L)])
    _unrolled(N // _L, _dinv_step)

    seq = [(p, ch) for p in range(_PASSES) for ch in range(nch)]
    bufs = [(cb0, wb0, rb0, sem0), (cb1, wb1, rb1, sem1)]

    def _issue(i):
        p, ch = seq[i]
        cb, wb, rb, sem = bufs[i % 2]
        sl = pl.ds(ch * _ECHUNK, _ECHUNK)
        return [pltpu.async_copy(ei_hbm.at[1, sl], cb, sem),
                pltpu.async_copy(ew_hbm.at[sl], wb, sem),
                pltpu.async_copy(ei_hbm.at[0, sl], rb, sem)]

    pending = {0: _issue(0)}
    for i, (p, ch) in enumerate(seq):
        base = (p * _NW + wid) * _ROWS_PER_TILE
        if ch == 0:
            def _zchunk(k):
                a_chunk[pl.ds(k * _L, _L)] = jnp.zeros((_L,), jnp.float32)
            _unrolled((_ROWS_PER_TILE * N) // _L, _zchunk, unroll=8)
        if i + 1 < len(seq):
            pending[(i + 1) % 2] = _issue(i + 1)
        for h in pending[i % 2]:
            h.wait()
        cb, wb, rb, _ = bufs[i % 2]

        def _edge_step(k):
            c16 = cb[pl.ds(k * _L, _L)]
            r16 = rb[pl.ds(k * _L, _L)]
            w16 = wb[pl.ds(k * _L, _L)]
            m = (c16 >= base) & (c16 < base + _ROWS_PER_TILE)
            dr = plsc.load_gather(dinv_l, [r16])
            dc = plsc.load_gather(dinv_l, [c16])
            val = dr * w16 * dc
            idx = jnp.where(m, (c16 - base) * N + r16, 0)
            plsc.addupdate_scatter(a_chunk, [idx], val, mask=m)
        _unrolled(_ECHUNK // _L, _edge_step, unroll=8)

        if ch == nch - 1:
            for t in range(_ROWS_PER_TILE // _L):
                j16 = base + t * _L + lax.iota(jnp.int32, _L)
                d16 = plsc.load_gather(dinv_l, [j16])
                idx = (j16 - base) * N + j16
                plsc.addupdate_scatter(a_chunk, [idx], d16 * d16)
            pltpu.sync_copy(
                a_chunk, a_hbm.at[pl.ds(base * N, _ROWS_PER_TILE * N)])


def _build_adj(edge_index, edge_weight):
    mesh = plsc.VectorSubcoreMesh(core_axis_name="c", subcore_axis_name="s")
    f = pl.kernel(
        _adj_body,
        out_type=jax.ShapeDtypeStruct((N * N,), jnp.float32),
        mesh=mesh,
        compiler_params=pltpu.CompilerParams(needs_layout_passes=False),
        scratch_types=[
            pltpu.VMEM((N,), jnp.float32),
            pltpu.VMEM((N,), jnp.float32),
            pltpu.VMEM((_ROWS_PER_TILE * N,), jnp.float32),
            pltpu.VMEM((_ECHUNK,), jnp.int32),
            pltpu.VMEM((_ECHUNK,), jnp.float32),
            pltpu.VMEM((_ECHUNK,), jnp.int32),
            pltpu.VMEM((_ECHUNK,), jnp.int32),
            pltpu.VMEM((_ECHUNK,), jnp.float32),
            pltpu.VMEM((_ECHUNK,), jnp.int32),
            pltpu.SemaphoreType.DMA,
            pltpu.SemaphoreType.DMA,
        ],
    )
    return f(edge_index, edge_weight).reshape(N, N)


def _h1_body(a_ref, x_ref, w_ref, b_ref, o_ref):
    t = lax.dot_general(a_ref[:, :], x_ref[:, :], (((1,), (0,)), ((), ())))
    t = lax.dot_general(t, w_ref[:, :], (((1,), (1,)), ((), ())))
    o_ref[:, :] = jnp.tanh(t + b_ref[:, :]).astype(jnp.bfloat16)


def _fused_body(h1_ref, wq_ref, wqb_ref, emb_ref, w1_ref, w2_ref, wfb_ref,
                o_ref):
    q = lax.dot_general(
        h1_ref[:, :], wq_ref[:, :].astype(jnp.bfloat16),
        (((1,), (1,)), ((), ())),
        preferred_element_type=jnp.float32)
    t = jnp.tanh(q + wqb_ref[0])
    f = jnp.sum(t * w1_ref[:, 0:1], axis=0, keepdims=True)
    g = lax.dot_general(
        w2_ref[:, :], emb_ref[:, :], (((1,), (1,)), ((), ())))
    o_ref[0] = f + g + wfb_ref[:, :]


def kernel(x, edge_index, edge_weight, gcn_W, gcn_b, Wq_W, Wq_b, emb, WF_W,
           WF_b):
    adj = _build_adj(edge_index, edge_weight)

    w1 = jnp.broadcast_to(WF_W[:, :N].reshape(N, 1), (N, 8))
    w2 = WF_W[:, N:]
    wqb = Wq_b.reshape(E // _EB, 1, _EB)

    h1 = pl.pallas_call(
        _h1_body,
        grid=(N // _RB,),
        in_specs=[
            pl.BlockSpec((_RB, N), lambda i: (i, 0)),
            pl.BlockSpec((N, N), lambda i: (0, 0)),
            pl.BlockSpec((N, N), lambda i: (0, 0)),
            pl.BlockSpec((1, N), lambda i: (0, 0)),
        ],
        out_specs=pl.BlockSpec((_RB, N), lambda i: (i, 0)),
        out_shape=jax.ShapeDtypeStruct((N, N), jnp.bfloat16),
    )(adj, x, gcn_W, gcn_b.reshape(1, N))

    out = pl.pallas_call(
        _fused_body,
        grid=(E // _EB,),
        in_specs=[
            pl.BlockSpec((N, N), lambda i: (0, 0)),
            pl.BlockSpec((_EB, N), lambda i: (i, 0)),
            pl.BlockSpec((1, 1, _EB), lambda i: (i, 0, 0)),
            pl.BlockSpec((_EB, D), lambda i: (i, 0)),
            pl.BlockSpec((N, 8), lambda i: (0, 0)),
            pl.BlockSpec((1, D), lambda i: (0, 0)),
            pl.BlockSpec((1, 1), lambda i: (0, 0)),
        ],
        out_specs=pl.BlockSpec((1, 1, _EB), lambda i: (i, 0, 0)),
        out_shape=jax.ShapeDtypeStruct((E // _EB, 1, _EB), jnp.float32),
        compiler_params=pltpu.CompilerParams(
            vmem_limit_bytes=63 * 1024 * 1024),
    )(h1, Wq_W, wqb, emb, w1, w2, WF_b.reshape(1, 1))

    return out.reshape(E)
